# baseline (speedup 1.0000x reference)
import jax
import jax.numpy as jnp
from jax.experimental import pallas as pl
from jax.experimental.pallas import tpu as pltpu

N = 10000
D = 128
B = 64
DC = 16
HG = 128
HC = 64
HF = 128

CHUNKS = (2560, 2816, 2688, 1936)
OFFS = (0, 2560, 5376, 8064)
NC = len(CHUNKS)

_DNT = (((1,), (1,)), ((), ()))


def _matT(a, w):
    return jax.lax.dot_general(a, w, _DNT, preferred_element_type=jnp.float32)


def _body(x_ref, b_ref, wg1_ref, bg1_ref, wg2_ref, bg2_ref,
          xct_ref, wc1t_ref, bc1_ref, wc2_ref, bc2_ref,
          wf1t_ref, bf1_ref, wf2_ref, bf2_ref,
          out_ref, *scratch):
    bufs, sem = scratch[:NC], scratch[NC]
    copies = []
    for k in range(NC):
        cp = pltpu.make_async_copy(
            x_ref.at[pl.ds(OFFS[k], CHUNKS[k]), :],
            bufs[k],
            sem.at[k])
        cp.start()
        copies.append(cp)

    c = jax.lax.dot_general(xct_ref[...], wc1t_ref[...],
                            (((0,), (0,)), ((), ())),
                            preferred_element_type=jnp.float32)
    c = jnp.maximum(c + bc1_ref[...], 0.0)
    c = _matT(c, wc2_ref[...])
    c = jnp.maximum(c + bc2_ref[...], 0.0)
    zc = jnp.dot(c, wf1t_ref[HG:HG + HC, :], preferred_element_type=jnp.float32)

    wg1 = wg1_ref[...].astype(jnp.bfloat16)
    wg2 = wg2_ref[...].astype(jnp.bfloat16)

    sums = jnp.zeros((B, HG), jnp.float32)
    cnt = jnp.zeros((B, 1), jnp.float32)
    for k in range(NC):
        ch = CHUNKS[k]
        copies[k].wait()
        xb = bufs[k][...].astype(jnp.bfloat16)
        h = _matT(xb, wg1)
        h = jnp.maximum(h + bg1_ref[...], 0.0).astype(jnp.bfloat16)
        h = _matT(h, wg2)
        h = jnp.maximum(h + bg2_ref[...], 0.0).astype(jnp.bfloat16)
        seg = b_ref[pl.ds(OFFS[k], ch)].reshape(1, ch)
        rows = jax.lax.broadcasted_iota(jnp.int32, (B, ch), 0)
        oh = (rows == seg).astype(jnp.bfloat16)
        sums += jnp.dot(oh, h, preferred_element_type=jnp.float32)
        cnt += jnp.sum(oh.astype(jnp.float32), axis=1, keepdims=True)

    pooled = sums / jnp.maximum(cnt, 1.0)
    z = zc + jnp.dot(pooled, wf1t_ref[0:HG, :],
                     preferred_element_type=jnp.float32)
    z = jnp.maximum(z + bf1_ref[...], 0.0)
    out_ref[...] = _matT(z, wf2_ref[...]) + bf2_ref[...]


@jax.jit
def kernel(x, x_context, edge_index, batch, Wg1, bg1, Wg2, bg2,
           Wc1, bc1, Wc2, bc2, Wf1, bf1, Wf2, bf2):
    del edge_index
    full = lambda shape: pl.BlockSpec(shape, lambda: (0,) * len(shape))
    out = pl.pallas_call(
        _body,
        grid=(),
        in_specs=[
            pl.BlockSpec(memory_space=pl.ANY),
            full((N,)),
            full((HG, D)), full((1, HG)),
            full((HG, HG)), full((1, HG)),
            full((DC, B)), full((DC, HC)), full((1, HC)),
            full((HC, HC)), full((1, HC)),
            full((HG + HC, HF)), full((1, HF)),
            full((HF, HF)), full((1, HF)),
        ],
        out_specs=pl.BlockSpec((B, HF), lambda: (0, 0)),
        out_shape=jax.ShapeDtypeStruct((B, HF), jnp.float32),
        scratch_shapes=[pltpu.VMEM((c, D), jnp.float32) for c in CHUNKS]
                       + [pltpu.SemaphoreType.DMA((NC,))],
    )(x, batch,
      Wg1, bg1[None, :], Wg2, bg2[None, :],
      x_context.T, Wc1.T, bc1[None, :], Wc2, bc2[None, :],
      Wf1.T, bf1[None, :], Wf2, bf2[None, :])
    return out


# speedup vs baseline: 1.0080x; 1.0080x over previous
import jax
import jax.numpy as jnp
from jax.experimental import pallas as pl
from jax.experimental.pallas import tpu as pltpu

N = 10000
D = 128
B = 64
DC = 16
HG = 128
HC = 64
HF = 128

CHUNKS = (3456, 3456, 2432, 656)
OFFS = (0, 3456, 6912, 9344)
NC = len(CHUNKS)

_DNT = (((1,), (1,)), ((), ()))


def _matT(a, w):
    return jax.lax.dot_general(a, w, _DNT, preferred_element_type=jnp.float32)


def _body(x_ref, b_ref, wg1_ref, bg1_ref, wg2_ref, bg2_ref,
          xct_ref, wc1t_ref, bc1_ref, wc2_ref, bc2_ref,
          wf1t_ref, bf1_ref, wf2_ref, bf2_ref,
          out_ref, *scratch):
    bufs, sem = scratch[:NC], scratch[NC]
    copies = []
    for k in range(NC):
        cp = pltpu.make_async_copy(
            x_ref.at[pl.ds(OFFS[k], CHUNKS[k]), :],
            bufs[k],
            sem.at[k])
        cp.start()
        copies.append(cp)

    c = jax.lax.dot_general(xct_ref[...], wc1t_ref[...],
                            (((0,), (0,)), ((), ())),
                            preferred_element_type=jnp.float32)
    c = jnp.maximum(c + bc1_ref[...], 0.0)
    c = _matT(c, wc2_ref[...])
    c = jnp.maximum(c + bc2_ref[...], 0.0)
    zc = jnp.dot(c, wf1t_ref[HG:HG + HC, :], preferred_element_type=jnp.float32)

    wg1 = wg1_ref[...].astype(jnp.bfloat16)
    wg2 = wg2_ref[...].astype(jnp.bfloat16)

    sums = jnp.zeros((B, HG), jnp.float32)
    cnt = jnp.zeros((B, 1), jnp.float32)
    for k in range(NC):
        ch = CHUNKS[k]
        copies[k].wait()
        xb = bufs[k][...].astype(jnp.bfloat16)
        h = _matT(xb, wg1)
        h = jnp.maximum(h + bg1_ref[...], 0.0).astype(jnp.bfloat16)
        h = _matT(h, wg2)
        h = jnp.maximum(h + bg2_ref[...], 0.0).astype(jnp.bfloat16)
        seg = b_ref[pl.ds(OFFS[k], ch)].reshape(1, ch)
        rows = jax.lax.broadcasted_iota(jnp.int32, (B, ch), 0)
        oh = (rows == seg).astype(jnp.bfloat16)
        sums += jnp.dot(oh, h, preferred_element_type=jnp.float32)
        cnt += jnp.sum(oh.astype(jnp.float32), axis=1, keepdims=True)

    pooled = sums / jnp.maximum(cnt, 1.0)
    z = zc + jnp.dot(pooled, wf1t_ref[0:HG, :],
                     preferred_element_type=jnp.float32)
    z = jnp.maximum(z + bf1_ref[...], 0.0)
    out_ref[...] = _matT(z, wf2_ref[...]) + bf2_ref[...]


@jax.jit
def kernel(x, x_context, edge_index, batch, Wg1, bg1, Wg2, bg2,
           Wc1, bc1, Wc2, bc2, Wf1, bf1, Wf2, bf2):
    del edge_index
    full = lambda shape: pl.BlockSpec(shape, lambda: (0,) * len(shape))
    out = pl.pallas_call(
        _body,
        grid=(),
        in_specs=[
            pl.BlockSpec(memory_space=pl.ANY),
            full((N,)),
            full((HG, D)), full((1, HG)),
            full((HG, HG)), full((1, HG)),
            full((DC, B)), full((DC, HC)), full((1, HC)),
            full((HC, HC)), full((1, HC)),
            full((HG + HC, HF)), full((1, HF)),
            full((HF, HF)), full((1, HF)),
        ],
        out_specs=pl.BlockSpec((B, HF), lambda: (0, 0)),
        out_shape=jax.ShapeDtypeStruct((B, HF), jnp.float32),
        scratch_shapes=[pltpu.VMEM((c, D), jnp.float32) for c in CHUNKS]
                       + [pltpu.SemaphoreType.DMA((NC,))],
    )(x, batch,
      Wg1, bg1[None, :], Wg2, bg2[None, :],
      x_context.T, Wc1.T, bc1[None, :], Wc2, bc2[None, :],
      Wf1.T, bf1[None, :], Wf2, bf2[None, :])
    return out


# speedup vs baseline: 1.0263x; 1.0181x over previous
import jax
import jax.numpy as jnp
from jax.experimental import pallas as pl
from jax.experimental.pallas import tpu as pltpu

N = 10000
D = 128
B = 64
DC = 16
HG = 128
HC = 64
HF = 128

CHUNKS = (3328, 3328, 3344)
OFFS = (0, 3328, 6656)
NC = len(CHUNKS)

_DNT = (((1,), (1,)), ((), ()))


def _matT(a, w):
    return jax.lax.dot_general(a, w, _DNT, preferred_element_type=jnp.float32)


def _body(x_ref, b_ref, wg1_ref, bg1_ref, wg2_ref, bg2_ref,
          xct_ref, wc1t_ref, bc1_ref, wc2_ref, bc2_ref,
          wf1t_ref, bf1_ref, wf2_ref, bf2_ref,
          out_ref, *scratch):
    bufs, sem = scratch[:NC], scratch[NC]
    copies = []
    for k in range(NC):
        cp = pltpu.make_async_copy(
            x_ref.at[pl.ds(OFFS[k], CHUNKS[k]), :],
            bufs[k],
            sem.at[k])
        cp.start()
        copies.append(cp)

    c = jax.lax.dot_general(xct_ref[...], wc1t_ref[...],
                            (((0,), (0,)), ((), ())),
                            preferred_element_type=jnp.float32)
    c = jnp.maximum(c + bc1_ref[...], 0.0)
    c = _matT(c, wc2_ref[...])
    c = jnp.maximum(c + bc2_ref[...], 0.0)
    zc = jnp.dot(c, wf1t_ref[HG:HG + HC, :], preferred_element_type=jnp.float32)

    wg1 = wg1_ref[...].astype(jnp.bfloat16)
    wg2 = wg2_ref[...].astype(jnp.bfloat16)

    sums = jnp.zeros((B, HG), jnp.float32)
    cnt = jnp.zeros((B, 1), jnp.float32)
    for k in range(NC):
        ch = CHUNKS[k]
        copies[k].wait()
        xb = bufs[k][...].astype(jnp.bfloat16)
        h = _matT(xb, wg1)
        h = jnp.maximum(h + bg1_ref[...], 0.0).astype(jnp.bfloat16)
        h = _matT(h, wg2)
        h = jnp.maximum(h + bg2_ref[...], 0.0).astype(jnp.bfloat16)
        seg = b_ref[pl.ds(OFFS[k], ch)].reshape(1, ch)
        rows = jax.lax.broadcasted_iota(jnp.int32, (B, ch), 0)
        oh = (rows == seg).astype(jnp.bfloat16)
        sums += jnp.dot(oh, h, preferred_element_type=jnp.float32)
        cnt += jnp.sum(oh.astype(jnp.float32), axis=1, keepdims=True)

    pooled = sums / jnp.maximum(cnt, 1.0)
    z = zc + jnp.dot(pooled, wf1t_ref[0:HG, :],
                     preferred_element_type=jnp.float32)
    z = jnp.maximum(z + bf1_ref[...], 0.0)
    out_ref[...] = _matT(z, wf2_ref[...]) + bf2_ref[...]


@jax.jit
def kernel(x, x_context, edge_index, batch, Wg1, bg1, Wg2, bg2,
           Wc1, bc1, Wc2, bc2, Wf1, bf1, Wf2, bf2):
    del edge_index
    full = lambda shape: pl.BlockSpec(shape, lambda: (0,) * len(shape))
    out = pl.pallas_call(
        _body,
        grid=(),
        in_specs=[
            pl.BlockSpec(memory_space=pl.ANY),
            full((N,)),
            full((HG, D)), full((1, HG)),
            full((HG, HG)), full((1, HG)),
            full((DC, B)), full((DC, HC)), full((1, HC)),
            full((HC, HC)), full((1, HC)),
            full((HG + HC, HF)), full((1, HF)),
            full((HF, HF)), full((1, HF)),
        ],
        out_specs=pl.BlockSpec((B, HF), lambda: (0, 0)),
        out_shape=jax.ShapeDtypeStruct((B, HF), jnp.float32),
        scratch_shapes=[pltpu.VMEM((c, D), jnp.float32) for c in CHUNKS]
                       + [pltpu.SemaphoreType.DMA((NC,))],
    )(x, batch,
      Wg1, bg1[None, :], Wg2, bg2[None, :],
      x_context.T, Wc1.T, bc1[None, :], Wc2, bc2[None, :],
      Wf1.T, bf1[None, :], Wf2, bf2[None, :])
    return out
